# baseline (speedup 1.0000x reference)
import jax
import jax.numpy as jnp
from jax.experimental import pallas as pl
from jax.experimental.pallas import tpu as pltpu

_D = 3
_N = 256
_H1 = 32
_H2 = 64
_INV = 1.0 / (_N - 1)


def _gram_diag_row(G):
    eye = (jax.lax.broadcasted_iota(jnp.int32, (_N, _N), 0) ==
           jax.lax.broadcasted_iota(jnp.int32, (_N, _N), 1))
    return jnp.sum(jnp.where(eye, G, 0.0), axis=0, keepdims=True)


def _nearest_onehot(G, sq_row):
    dred = sq_row - 2.0 * G
    minv = jnp.min(dred, axis=1, keepdims=True)
    lane = jax.lax.broadcasted_iota(jnp.int32, (_N, _N), 1)
    m_col = jnp.min(jnp.where(dred == minv, lane, _N),
                    axis=1, keepdims=True)
    return (lane == m_col).astype(jnp.float32)


def _relu_layer(h, W, b_row, d_out, cd):
    f32 = jnp.float32
    d_x = h.shape[cd]
    WL = W[:, :d_x]
    WR = W[:, d_x:]

    G = jax.lax.dot_general(h, h, (((cd,), (cd,)), ((), ())),
                            preferred_element_type=f32)
    P = _nearest_onehot(G, _gram_diag_row(G))

    A = jax.lax.dot_general(h, WL, (((cd,), (1,)), ((), ())),
                            preferred_element_type=f32) + b_row
    C = jax.lax.dot_general(h, WR, (((cd,), (1,)), ((), ())),
                            preferred_element_type=f32)
    Cm = jax.lax.dot_general(P, C, (((1,), (0,)), ((), ())),
                             preferred_element_type=f32)

    p = 128 // d_out
    nb = _N // p
    Cpack = jnp.concatenate([C[s * nb:(s + 1) * nb, :] for s in range(p)],
                            axis=1)
    Apack = jnp.concatenate([A] * p, axis=1)
    acc = jnp.maximum(Apack + Cpack[0:1, :], 0.0)
    for jj in range(1, nb):
        acc = acc + jnp.maximum(Apack + Cpack[jj:jj + 1, :], 0.0)
    S = acc[:, :d_out]
    for s in range(1, p):
        S = S + acc[:, s * d_out:(s + 1) * d_out]
    return (S - jnp.maximum(A + Cm, 0.0)) * _INV


def _final_layer_T(h, W, b_col):
    f32 = jnp.float32
    WL = W[:, :_H2]
    WR = W[:, _H2:]
    G = jax.lax.dot_general(h, h, (((1,), (1,)), ((), ())),
                            preferred_element_type=f32)
    P = _nearest_onehot(G, _gram_diag_row(G))
    A_T = jax.lax.dot_general(WL, h, (((1,), (1,)), ((), ())),
                              preferred_element_type=f32) + b_col
    C_T = jax.lax.dot_general(WR, h, (((1,), (1,)), ((), ())),
                              preferred_element_type=f32)
    sumC = jnp.sum(C_T, axis=1, keepdims=True)
    Cm_T = jax.lax.dot_general(C_T, P, (((1,), (1,)), ((), ())),
                               preferred_element_type=f32)
    return A_T + (sumC - Cm_T) * _INV


_BPP = 8


def _net_kernel(x_ref, W1_ref, b1_ref, W2_ref, b2_ref, W3_ref, b3_ref, out_ref):
    for i in range(_BPP):
        h = _relu_layer(x_ref[i], W1_ref[...], b1_ref[...], _H1, cd=0)
        h = _relu_layer(h, W2_ref[...], b2_ref[...], _H2, cd=1)
        out_ref[i] = _final_layer_T(h, W3_ref[...], b3_ref[...])


def kernel(x, W1, b1, W2, b2, W3, b3):
    B = x.shape[0]
    x_T = x.reshape(B, _N, _D).transpose(0, 2, 1)
    out = pl.pallas_call(
        _net_kernel,
        grid=(1,),
        in_specs=[
            pl.BlockSpec((_BPP, _D, _N), lambda b: (0, 0, 0)),
            pl.BlockSpec(W1.shape, lambda b: (0, 0)),
            pl.BlockSpec((1, _H1), lambda b: (0, 0)),
            pl.BlockSpec(W2.shape, lambda b: (0, 0)),
            pl.BlockSpec((1, _H2), lambda b: (0, 0)),
            pl.BlockSpec(W3.shape, lambda b: (0, 0)),
            pl.BlockSpec((_D, 1), lambda b: (0, 0)),
        ],
        out_specs=pl.BlockSpec((_BPP, _D, _N), lambda b: (0, 0, 0)),
        out_shape=jax.ShapeDtypeStruct((B, _D, _N), jnp.float32),
        compiler_params=pltpu.CompilerParams(
            dimension_semantics=("arbitrary",)),
    )(x_T, W1, b1.reshape(1, _H1), W2, b2.reshape(1, _H2),
      W3, b3.reshape(_D, 1))
    return out.transpose(0, 2, 1).reshape(B, _N * _D)


# speedup vs baseline: 1.0083x; 1.0083x over previous
import jax
import jax.numpy as jnp
from jax.experimental import pallas as pl
from jax.experimental.pallas import tpu as pltpu

_D = 3
_N = 256
_H1 = 32
_H2 = 64
_INV = 1.0 / (_N - 1)


def _gram_diag_row(G):
    eye = (jax.lax.broadcasted_iota(jnp.int32, (_N, _N), 0) ==
           jax.lax.broadcasted_iota(jnp.int32, (_N, _N), 1))
    return jnp.sum(jnp.where(eye, G, 0.0), axis=0, keepdims=True)


def _nearest_onehot(G, sq_row):
    dred = sq_row - 2.0 * G
    minv = jnp.min(dred, axis=1, keepdims=True)
    lane = jax.lax.broadcasted_iota(jnp.int32, (_N, _N), 1)
    m_col = jnp.min(jnp.where(dred == minv, lane, _N),
                    axis=1, keepdims=True)
    return (lane == m_col).astype(jnp.float32)


def _relu_layer(h, W, b_row, d_out, cd):
    f32 = jnp.float32
    d_x = h.shape[cd]
    WL = W[:, :d_x]
    WR = W[:, d_x:]

    G = jax.lax.dot_general(h, h, (((cd,), (cd,)), ((), ())),
                            preferred_element_type=f32)
    P = _nearest_onehot(G, _gram_diag_row(G))

    A = jax.lax.dot_general(h, WL, (((cd,), (1,)), ((), ())),
                            preferred_element_type=f32) + b_row
    C = jax.lax.dot_general(h, WR, (((cd,), (1,)), ((), ())),
                            preferred_element_type=f32)
    Cm = jax.lax.dot_general(P, C, (((1,), (0,)), ((), ())),
                             preferred_element_type=f32)

    p = 128 // d_out
    nb = _N // p
    Cpack = jnp.concatenate([C[s * nb:(s + 1) * nb, :] for s in range(p)],
                            axis=1)
    Apack = jnp.concatenate([A] * p, axis=1)
    acc0 = jnp.maximum(Apack + Cpack[0:1, :], 0.0)
    acc1 = jnp.maximum(Apack + Cpack[1:2, :], 0.0)
    for jj in range(2, nb, 2):
        acc0 = acc0 + jnp.maximum(Apack + Cpack[jj:jj + 1, :], 0.0)
        acc1 = acc1 + jnp.maximum(Apack + Cpack[jj + 1:jj + 2, :], 0.0)
    acc = acc0 + acc1
    S = acc[:, :d_out]
    for s in range(1, p):
        S = S + acc[:, s * d_out:(s + 1) * d_out]
    return (S - jnp.maximum(A + Cm, 0.0)) * _INV


def _final_layer_T(h, W, b_col):
    f32 = jnp.float32
    WL = W[:, :_H2]
    WR = W[:, _H2:]
    G = jax.lax.dot_general(h, h, (((1,), (1,)), ((), ())),
                            preferred_element_type=f32)
    P = _nearest_onehot(G, _gram_diag_row(G))
    A_T = jax.lax.dot_general(WL, h, (((1,), (1,)), ((), ())),
                              preferred_element_type=f32) + b_col
    C_T = jax.lax.dot_general(WR, h, (((1,), (1,)), ((), ())),
                              preferred_element_type=f32)
    sumC = jnp.sum(C_T, axis=1, keepdims=True)
    Cm_T = jax.lax.dot_general(C_T, P, (((1,), (1,)), ((), ())),
                               preferred_element_type=f32)
    return A_T + (sumC - Cm_T) * _INV


_BPP = 8


def _net_kernel(x_ref, W1_ref, b1_ref, W2_ref, b2_ref, W3_ref, b3_ref, out_ref):
    for i in range(_BPP):
        h = _relu_layer(x_ref[i], W1_ref[...], b1_ref[...], _H1, cd=0)
        h = _relu_layer(h, W2_ref[...], b2_ref[...], _H2, cd=1)
        out_ref[i] = _final_layer_T(h, W3_ref[...], b3_ref[...])


def kernel(x, W1, b1, W2, b2, W3, b3):
    B = x.shape[0]
    x_T = x.reshape(B, _N, _D).transpose(0, 2, 1)
    out = pl.pallas_call(
        _net_kernel,
        grid=(1,),
        in_specs=[
            pl.BlockSpec((_BPP, _D, _N), lambda b: (0, 0, 0)),
            pl.BlockSpec(W1.shape, lambda b: (0, 0)),
            pl.BlockSpec((1, _H1), lambda b: (0, 0)),
            pl.BlockSpec(W2.shape, lambda b: (0, 0)),
            pl.BlockSpec((1, _H2), lambda b: (0, 0)),
            pl.BlockSpec(W3.shape, lambda b: (0, 0)),
            pl.BlockSpec((_D, 1), lambda b: (0, 0)),
        ],
        out_specs=pl.BlockSpec((_BPP, _D, _N), lambda b: (0, 0, 0)),
        out_shape=jax.ShapeDtypeStruct((B, _D, _N), jnp.float32),
        compiler_params=pltpu.CompilerParams(
            dimension_semantics=("arbitrary",)),
    )(x_T, W1, b1.reshape(1, _H1), W2, b2.reshape(1, _H2),
      W3, b3.reshape(_D, 1))
    return out.transpose(0, 2, 1).reshape(B, _N * _D)
